# baseline (speedup 1.0000x reference)
import itertools

import jax
import jax.numpy as jnp
from jax.experimental import pallas as pl
from jax.experimental.pallas import tpu as pltpu

_AREAS = (4096, 1024, 256, 64, 16)
_OFFS = (0, 4096, 5120, 5376, 5440)
_A = 5456
_NC = 80
_MAXOBJ = 10


def _fused_kernel(cnt_ref, cls_pred_ref, cls_tar_ref, loc_pred_ref,
                  loc_tar_ref, ind_ref, out_ref):
    cp = cls_pred_ref[0]
    ct = cls_tar_ref[0]
    t = ct[:, :_NC]

    p = jnp.clip(jax.nn.sigmoid(cp), 1e-7, 1.0 - 1e-7)
    ce = -(t * jnp.log(p) + (1.0 - t) * jnp.log(1.0 - p))
    a_t = t * 0.25 + (1.0 - t) * 0.75
    p_t = t * p + (1.0 - t) * (1.0 - p)
    om = 1.0 - p_t
    cls_loss = jnp.sum(a_t * om * om * ce, axis=-1, keepdims=True)

    lp = loc_pred_ref[0]
    lt = loc_tar_ref[0]
    pl_, pt_, pr_, pb_ = lp[:, 0:1], lp[:, 1:2], lp[:, 2:3], lp[:, 3:4]
    tl_, tt_, tr_, tb_ = lt[:, 0:1], lt[:, 1:2], lt[:, 2:3], lt[:, 3:4]
    area_p = (pl_ + pr_) * (pt_ + pb_)
    area_t = (tl_ + tr_) * (tt_ + tb_)
    iw = jnp.minimum(pl_, tl_) + jnp.minimum(pr_, tr_)
    ih = jnp.minimum(pt_, tt_) + jnp.minimum(pb_, tb_)
    inter = jnp.maximum(iw, 0.0) * jnp.maximum(ih, 0.0)
    union = area_p + area_t - inter + 1e-7
    iou = inter / union
    cw = jnp.maximum(pl_, tl_) + jnp.maximum(pr_, tr_)
    ch = jnp.maximum(pt_, tt_) + jnp.maximum(pb_, tb_)
    area_c = cw * ch + 1e-7
    loc_loss = 1.0 - (iou - (area_c - union) / area_c)

    total = cls_loss + loc_loss

    ind = ind_ref[0]
    oids = jax.lax.broadcasted_iota(jnp.int32, (_A, _MAXOBJ), 1)
    onehot = (ind == oids).astype(jnp.float32)
    w = total * onehot
    sums, cnts = [], []
    for off, a in zip(_OFFS, _AREAS):
        sums.append(jnp.sum(w[off:off + a], axis=0, keepdims=True))
        cnts.append(jnp.sum(onehot[off:off + a], axis=0, keepdims=True))
    S = jnp.concatenate(sums, axis=0)
    C = jnp.concatenate(cnts, axis=0)

    mean = S / jnp.maximum(1.0, C)
    lmax = jnp.max(mean, axis=0, keepdims=True) + 1e-5
    mean = jnp.where(mean == 0.0, lmax, mean)
    lmin = jnp.min(mean, axis=0, keepdims=True)
    tgt = 1.0 - (mean - lmin) / jnp.maximum(lmax - lmin, 1e-12)

    rows = [tgt[i:i + 1] for i in range(5)]
    min_w = None
    for i, j, k in itertools.combinations(range(5), 3):
        t3 = jnp.minimum(jnp.minimum(rows[i], rows[j]), rows[k])
        min_w = t3 if min_w is None else jnp.maximum(min_w, t3)
    tgt = jnp.where(tgt >= min_w, tgt, 0.0)

    cnt = cnt_ref[pl.program_id(0), 0]
    gate = (jax.lax.broadcasted_iota(jnp.int32, (1, _MAXOBJ), 1)
            < cnt).astype(jnp.float32)
    tgt = tgt * gate

    dws = []
    for l, (off, a) in enumerate(zip(_OFFS, _AREAS)):
        dws.append(jnp.sum(onehot[off:off + a] * tgt[l:l + 1],
                           axis=1, keepdims=True))
    dwlm = jnp.concatenate(dws, axis=0)

    mask = ct[:, _NC + 1:_NC + 2]
    out_ref[0] = jnp.where(mask > 0.0, dwlm, 1.0)


def kernel(cls_pred, loc_pred, cls_tar, loc_tar, ind_tar, bboxes_cnt):
    B = cls_pred.shape[0]
    out = pl.pallas_call(
        _fused_kernel,
        grid=(B,),
        in_specs=[
            pl.BlockSpec(memory_space=pltpu.SMEM),
            pl.BlockSpec((1, _A, _NC), lambda b: (b, 0, 0)),
            pl.BlockSpec((1, _A, _NC + 2), lambda b: (b, 0, 0)),
            pl.BlockSpec((1, _A, 4), lambda b: (b, 0, 0)),
            pl.BlockSpec((1, _A, 4), lambda b: (b, 0, 0)),
            pl.BlockSpec((1, _A, 1), lambda b: (b, 0, 0)),
        ],
        out_specs=pl.BlockSpec((1, _A, 1), lambda b: (b, 0, 0)),
        out_shape=jax.ShapeDtypeStruct((B, _A, 1), jnp.float32),
    )(bboxes_cnt, cls_pred, cls_tar, loc_pred, loc_tar, ind_tar)
    mask = cls_tar[..., -1]
    return (out, mask)


# speedup vs baseline: 2.6792x; 2.6792x over previous
import itertools

import jax
import jax.numpy as jnp
from jax.experimental import pallas as pl
from jax.experimental.pallas import tpu as pltpu

_AREAS = (4096, 1024, 256, 64, 16)
_OFFS = (0, 4096, 5120, 5376, 5440)
_A = 5456
_NC = 80
_MAXOBJ = 10


def _fused_kernel(cnt_ref, cls_pred_ref, cls_tar_ref, loc_pred_ref,
                  loc_tar_ref, ind_ref, mask_ref, out_ref):
    x = jnp.transpose(cls_pred_ref[0])
    t = jnp.transpose(cls_tar_ref[0][:, :_NC])

    p = jnp.clip(jax.nn.sigmoid(x), 1e-7, 1.0 - 1e-7)
    lp = jnp.log(p)
    lq = jnp.log(1.0 - p)
    ce = -(t * lp + (1.0 - t) * lq)
    a_t = 0.75 - 0.5 * t
    tp = 2.0 * p - 1.0
    om = p - t * tp
    f = a_t * om * om * ce
    cls_loss = jnp.sum(f, axis=0, keepdims=True)

    lpd = loc_pred_ref[0]
    ltd = loc_tar_ref[0]
    pl_, pt_, pr_, pb_ = (lpd[0:1], lpd[1:2], lpd[2:3], lpd[3:4])
    tl_, tt_, tr_, tb_ = (ltd[0:1], ltd[1:2], ltd[2:3], ltd[3:4])
    area_p = (pl_ + pr_) * (pt_ + pb_)
    area_t = (tl_ + tr_) * (tt_ + tb_)
    iw = jnp.minimum(pl_, tl_) + jnp.minimum(pr_, tr_)
    ih = jnp.minimum(pt_, tt_) + jnp.minimum(pb_, tb_)
    inter = jnp.maximum(iw, 0.0) * jnp.maximum(ih, 0.0)
    union = area_p + area_t - inter + 1e-7
    iou = inter / union
    cw = jnp.maximum(pl_, tl_) + jnp.maximum(pr_, tr_)
    ch = jnp.maximum(pt_, tt_) + jnp.maximum(pb_, tb_)
    area_c = cw * ch + 1e-7
    loc_loss = 1.0 - (iou - (area_c - union) / area_c)

    total = cls_loss + loc_loss

    ind = ind_ref[0]
    onehots = []
    s_rows, c_rows = [], []
    for o in range(_MAXOBJ):
        oh = (ind == o).astype(jnp.float32)
        onehots.append(oh)
        m = total * oh
        s_cells = []
        c_cells = []
        for off, a in zip(_OFFS, _AREAS):
            s_cells.append(jnp.sum(m[:, off:off + a], axis=1, keepdims=True))
            c_cells.append(jnp.sum(oh[:, off:off + a], axis=1, keepdims=True))
        s_rows.append(jnp.concatenate(s_cells, axis=1))
        c_rows.append(jnp.concatenate(c_cells, axis=1))
    S = jnp.concatenate(s_rows, axis=0)
    C = jnp.concatenate(c_rows, axis=0)

    mean = S / jnp.maximum(1.0, C)
    lmax = jnp.max(mean, axis=1, keepdims=True) + 1e-5
    mean = jnp.where(mean == 0.0, lmax, mean)
    lmin = jnp.min(mean, axis=1, keepdims=True)
    tgt = 1.0 - (mean - lmin) / jnp.maximum(lmax - lmin, 1e-12)

    cols = [tgt[:, i:i + 1] for i in range(5)]
    min_w = None
    for i, j, k in itertools.combinations(range(5), 3):
        t3 = jnp.minimum(jnp.minimum(cols[i], cols[j]), cols[k])
        min_w = t3 if min_w is None else jnp.maximum(min_w, t3)
    tgt = jnp.where(tgt >= min_w, tgt, 0.0)

    cnt = cnt_ref[pl.program_id(0), 0]
    gate = (jax.lax.broadcasted_iota(jnp.int32, (_MAXOBJ, 1), 0)
            < cnt).astype(jnp.float32)
    tgt = tgt * gate

    dwlm = jnp.zeros((1, _A), dtype=jnp.float32)
    for o in range(_MAXOBJ):
        tmap = jnp.concatenate(
            [jnp.broadcast_to(tgt[o:o + 1, l:l + 1], (1, a))
             for l, a in enumerate(_AREAS)], axis=1)
        dwlm = dwlm + onehots[o] * tmap

    mask = mask_ref[0]
    out_ref[0] = jnp.where(mask > 0.0, dwlm, 1.0)


def kernel(cls_pred, loc_pred, cls_tar, loc_tar, ind_tar, bboxes_cnt):
    B = cls_pred.shape[0]
    loc_pred_t = jnp.transpose(loc_pred, (0, 2, 1))
    loc_tar_t = jnp.transpose(loc_tar, (0, 2, 1))
    ind_t = ind_tar.reshape(B, 1, _A)
    mask = cls_tar[..., -1]
    mask3 = mask.reshape(B, 1, _A)
    out = pl.pallas_call(
        _fused_kernel,
        grid=(B,),
        in_specs=[
            pl.BlockSpec(memory_space=pltpu.SMEM),
            pl.BlockSpec((1, _A, _NC), lambda b: (b, 0, 0)),
            pl.BlockSpec((1, _A, _NC + 2), lambda b: (b, 0, 0)),
            pl.BlockSpec((1, 4, _A), lambda b: (b, 0, 0)),
            pl.BlockSpec((1, 4, _A), lambda b: (b, 0, 0)),
            pl.BlockSpec((1, 1, _A), lambda b: (b, 0, 0)),
            pl.BlockSpec((1, 1, _A), lambda b: (b, 0, 0)),
        ],
        out_specs=pl.BlockSpec((1, 1, _A), lambda b: (b, 0, 0)),
        out_shape=jax.ShapeDtypeStruct((B, 1, _A), jnp.float32),
    )(bboxes_cnt, cls_pred, cls_tar, loc_pred_t, loc_tar_t, ind_t, mask3)
    return (out.reshape(B, _A, 1), mask)
